# baseline (speedup 1.0000x reference)
import functools

import jax
import jax.numpy as jnp
from jax import lax
from jax.experimental import pallas as pl
from jax.experimental.pallas import tpu as pltpu
from jax.experimental.pallas import tpu_sc as plsc

B, N, C = 16, 131072, 3
NPOINT = 16384

TILES = 32
LANES = 16

ROW_TILE, COL_TILE = 8, 128
NT = N // COL_TILE
TILE_WORDS = ROW_TILE * COL_TILE
BT = B // ROW_TILE
RANGES = TILES // BT
NPT = NPOINT // COL_TILE
RPT = NPT // RANGES
PTS = RPT * ROW_TILE * COL_TILE
PLANE = BT * NT * TILE_WORDS
STEPS = PTS // LANES
GCHUNKS = 8


def _sc_gather(xyz_tiles, pidx_tiles):
    mesh = plsc.VectorSubcoreMesh(core_axis_name="c", subcore_axis_name="s")

    @functools.partial(
        pl.kernel,
        mesh=mesh,
        compiler_params=pltpu.CompilerParams(
            use_tc_tiling_on_sc=False,
            needs_layout_passes=False,
            skip_device_barrier=True,
        ),
        out_type=jax.ShapeDtypeStruct((C, BT, RANGES, PTS), jnp.float32),
        scratch_types=[
            pltpu.VMEM((RPT, ROW_TILE, COL_TILE), jnp.int32),
            pltpu.VMEM((PTS,), jnp.int32),
            pltpu.VMEM((PTS,), jnp.float32),
            pltpu.VMEM((PTS,), jnp.float32),
            pltpu.VMEM((PTS,), jnp.float32),
            pltpu.SemaphoreType.DMA,
            pltpu.SemaphoreType.DMA,
            pltpu.SemaphoreType.DMA,
        ],
    )
    def k(xyz_hbm, pidx_hbm, out_hbm, idx_v, a_v,
          p0_v, p1_v, p2_v, sem0, sem1, osem):
        wid = lax.axis_index("s") * 2 + lax.axis_index("c")
        bt = wid // RANGES
        r = wid % RANGES

        pltpu.sync_copy(pidx_hbm.at[bt, pl.ds(r * RPT, RPT)], idx_v)

        bt_base = bt * (NT * TILE_WORDS)

        def expand(kk, carry):
            nt_ = kk // 64
            rem = kk - nt_ * 64
            b8 = rem // 8
            i = rem - b8 * 8
            v = idx_v[nt_, b8, pl.ds(i * LANES, LANES)]
            t = ((v >> 7) << 10) + (v & 127) + (bt_base + b8 * COL_TILE)
            a_v[pl.ds(kk * LANES, LANES)] = t
            return carry

        planes = (p0_v, p1_v, p2_v)
        sems = (sem0, sem1)
        GSTEPS = STEPS // GCHUNKS
        GPTS = PTS // GCHUNKS
        copies = []
        for g in range(GCHUNKS):
            lax.fori_loop(g * GSTEPS, (g + 1) * GSTEPS, expand, 0)
            for c in range(C):
                copies.append(
                    pltpu.async_copy(
                        xyz_hbm.at[pl.ds(c * PLANE, PLANE)].at[
                            a_v.at[pl.ds(g * GPTS, GPTS)]
                        ],
                        planes[c].at[pl.ds(g * GPTS, GPTS)],
                        sems[g % 2],
                    )
                )
        for cp in copies:
            cp.wait()
        outs = [
            pltpu.async_copy(planes[c], out_hbm.at[c, bt, r], osem)
            for c in range(C)
        ]
        for cp in outs:
            cp.wait()

    return k(xyz_tiles, pidx_tiles)


def kernel(xyz, point_indices):
    xyz_tiles = (
        xyz.transpose(2, 0, 1)
        .reshape(C, BT, ROW_TILE, NT, COL_TILE)
        .transpose(0, 1, 3, 2, 4)
        .reshape(-1)
    )
    pidx_tiles = (
        point_indices.reshape(BT, ROW_TILE, NPT, COL_TILE)
        .transpose(0, 2, 1, 3)
    )
    out5 = _sc_gather(xyz_tiles, pidx_tiles)
    out = (
        out5.reshape(C, BT, NPT, ROW_TILE, COL_TILE)
        .transpose(1, 3, 2, 4, 0)
        .reshape(B, NPOINT, C)
    )
    return out


# speedup vs baseline: 1.0045x; 1.0045x over previous
import functools

import jax
import jax.numpy as jnp
from jax import lax
from jax.experimental import pallas as pl
from jax.experimental.pallas import tpu as pltpu
from jax.experimental.pallas import tpu_sc as plsc

B, N, C = 16, 131072, 3
NPOINT = 16384

TILES = 32
LANES = 16

ROW_TILE, COL_TILE = 8, 128
NT = N // COL_TILE
TILE_WORDS = ROW_TILE * COL_TILE
BT = B // ROW_TILE
RANGES = TILES // BT
NPT = NPOINT // COL_TILE
RPT = NPT // RANGES
PTS = RPT * ROW_TILE * COL_TILE
PLANE = BT * NT * TILE_WORDS
STEPS = PTS // LANES
GCHUNKS = 4


def _sc_gather(xyz_tiles, pidx_tiles):
    mesh = plsc.VectorSubcoreMesh(core_axis_name="c", subcore_axis_name="s")

    @functools.partial(
        pl.kernel,
        mesh=mesh,
        compiler_params=pltpu.CompilerParams(
            use_tc_tiling_on_sc=False,
            needs_layout_passes=False,
            skip_device_barrier=True,
        ),
        out_type=jax.ShapeDtypeStruct((C, BT, RANGES, PTS), jnp.float32),
        scratch_types=[
            pltpu.VMEM((RPT, ROW_TILE, COL_TILE), jnp.int32),
            pltpu.VMEM((PTS,), jnp.int32),
            pltpu.VMEM((PTS,), jnp.float32),
            pltpu.VMEM((PTS,), jnp.float32),
            pltpu.VMEM((PTS,), jnp.float32),
            pltpu.SemaphoreType.DMA,
            pltpu.SemaphoreType.DMA,
            pltpu.SemaphoreType.DMA,
        ],
    )
    def k(xyz_hbm, pidx_hbm, out_hbm, idx_v, a_v,
          p0_v, p1_v, p2_v, sem0, sem1, osem):
        wid = lax.axis_index("s") * 2 + lax.axis_index("c")
        bt = wid // RANGES
        r = wid % RANGES

        pltpu.sync_copy(pidx_hbm.at[bt, pl.ds(r * RPT, RPT)], idx_v)

        bt_base = bt * (NT * TILE_WORDS)

        def expand(kk, carry):
            nt_ = kk // 64
            rem = kk - nt_ * 64
            b8 = rem // 8
            i = rem - b8 * 8
            v = idx_v[nt_, b8, pl.ds(i * LANES, LANES)]
            t = ((v >> 7) << 10) + (v & 127) + (bt_base + b8 * COL_TILE)
            a_v[pl.ds(kk * LANES, LANES)] = t
            return carry

        planes = (p0_v, p1_v, p2_v)
        sems = (sem0, sem1)
        GSTEPS = STEPS // GCHUNKS
        GPTS = PTS // GCHUNKS
        copies = []
        for g in range(GCHUNKS):
            lax.fori_loop(g * GSTEPS, (g + 1) * GSTEPS, expand, 0)
            for c in range(C):
                copies.append(
                    pltpu.async_copy(
                        xyz_hbm.at[pl.ds(c * PLANE, PLANE)].at[
                            a_v.at[pl.ds(g * GPTS, GPTS)]
                        ],
                        planes[c].at[pl.ds(g * GPTS, GPTS)],
                        sems[g % 2],
                    )
                )
        for cp in copies:
            cp.wait()
        outs = [
            pltpu.async_copy(planes[c], out_hbm.at[c, bt, r], osem)
            for c in range(C)
        ]
        for cp in outs:
            cp.wait()

    return k(xyz_tiles, pidx_tiles)


def kernel(xyz, point_indices):
    xyz_tiles = (
        xyz.transpose(2, 0, 1)
        .reshape(C, BT, ROW_TILE, NT, COL_TILE)
        .transpose(0, 1, 3, 2, 4)
        .reshape(-1)
    )
    pidx_tiles = (
        point_indices.reshape(BT, ROW_TILE, NPT, COL_TILE)
        .transpose(0, 2, 1, 3)
    )
    out5 = _sc_gather(xyz_tiles, pidx_tiles)
    out = (
        out5.reshape(C, BT, NPT, ROW_TILE, COL_TILE)
        .transpose(1, 3, 2, 4, 0)
        .reshape(B, NPOINT, C)
    )
    return out
